# baseline (speedup 1.0000x reference)
import functools

import jax
import jax.numpy as jnp
from jax import lax
from jax.experimental import pallas as pl
from jax.experimental.pallas import tpu as pltpu
from jax.experimental.pallas import tpu_sc as plsc

NCORES = 2
NSUB = 16
NW = NCORES * NSUB
LANES = 16
CH = 128


def _mesh():
    return plsc.VectorSubcoreMesh(core_axis_name="c", subcore_axis_name="s",
                                  num_cores=NCORES, num_subcores=NSUB)


def _rsqrt16(x):
    t = 0.5 * (x + 1.0)
    for _ in range(12):
        t = 0.5 * (t + x / t)
    return 1.0 / t


def _make_sc_deg(n_pad, n_edges):
    ept = n_edges // NW
    B = 128
    nfull = ept // B
    rem = ept - nfull * B
    rpt = n_pad // NSUB
    nch = rpt // CH

    @functools.partial(
        pl.kernel,
        out_type=jax.ShapeDtypeStruct((NCORES * n_pad, LANES), jnp.float32),
        mesh=_mesh(),
        scratch_types=[
            pltpu.VMEM((B,), jnp.int32),
            pltpu.VMEM((rem,), jnp.int32),
            pltpu.VMEM((B, LANES), jnp.float32),
            pltpu.VMEM((CH, LANES), jnp.float32),
            pltpu.VMEM_SHARED((n_pad, LANES), jnp.float32),
        ],
    )
    def deg_kernel(dst_hbm, out_hbm, idx_v, idxr_v, ones_v, stage_v, acc_sh):
        cid = lax.axis_index("c")
        sid = lax.axis_index("s")
        wid = sid * NCORES + cid

        ones16 = jnp.ones((LANES,), jnp.float32)
        zero16 = jnp.zeros((LANES,), jnp.float32)

        def fill_ones(i, _):
            ones_v[i, :] = ones16
            return 0
        lax.fori_loop(0, B, fill_ones, 0)

        def fill_zero(i, _):
            stage_v[i, :] = zero16
            return 0
        lax.fori_loop(0, CH, fill_zero, 0)

        def zchunk(i, _):
            pltpu.sync_copy(stage_v, acc_sh.at[pl.ds(sid * rpt + i * CH, CH)])
            return 0
        lax.fori_loop(0, nch, zchunk, 0)
        plsc.subcore_barrier()

        base = wid * ept

        def batch(i, _):
            pltpu.sync_copy(dst_hbm.at[pl.ds(base + i * B, B)], idx_v)
            pltpu.sync_copy(ones_v, acc_sh.at[idx_v], add=True)
            return 0
        lax.fori_loop(0, nfull, batch, 0)
        if rem:
            pltpu.sync_copy(dst_hbm.at[pl.ds(base + nfull * B, rem)], idxr_v)
            pltpu.sync_copy(ones_v.at[pl.ds(0, rem)], acc_sh.at[idxr_v], add=True)

        plsc.subcore_barrier()

        def ochunk(i, _):
            r = sid * rpt + i * CH
            pltpu.sync_copy(acc_sh.at[pl.ds(r, CH)], stage_v)
            pltpu.sync_copy(stage_v, out_hbm.at[pl.ds(cid * n_pad + r, CH)])
            return 0
        lax.fori_loop(0, nch, ochunk, 0)

    return deg_kernel


def _make_sc_scatter(n_nodes, n_pad, n_edges, d):
    ept = n_edges // NW
    B = 128
    nfull = ept // B
    rem = ept - nfull * B
    rpt = n_pad // NSUB
    nch = rpt // CH
    ZCH = 64
    last_avail = n_nodes - (NSUB - 1) * rpt
    zfull, zpart = rpt // ZCH, rpt % ZCH
    lfull, lpart = last_avail // ZCH, last_avail % ZCH
    nvec = d // LANES

    @functools.partial(
        pl.kernel,
        out_type=[
            jax.ShapeDtypeStruct((NCORES * n_pad, d), jnp.float32),
            jax.ShapeDtypeStruct((NCORES * n_pad, d), jnp.float32),
        ],
        mesh=_mesh(),
        scratch_types=[
            pltpu.VMEM((B,), jnp.int32),
            pltpu.VMEM((B,), jnp.int32),
            pltpu.VMEM((rem,), jnp.int32),
            pltpu.VMEM((rem,), jnp.int32),
            pltpu.VMEM((B, d), jnp.float32),
            pltpu.VMEM((rem, d), jnp.float32),
            pltpu.VMEM((ZCH, LANES), jnp.float32),
            pltpu.VMEM((ZCH, LANES), jnp.float32),
            pltpu.VMEM_SHARED((n_pad, d), jnp.float32),
            pltpu.SemaphoreType.DMA,
        ],
    )
    def scat_kernel(eflat_hbm, x_hbm, deg_hbm, s_out, z_out,
                    sidx_v, didx_v, sidxr_v, didxr_v, rows_v, rowsr_v,
                    dg0_v, dg1_v, acc_sh, sem):
        cid = lax.axis_index("c")
        sid = lax.axis_index("s")
        wid = sid * NCORES + cid
        r0 = sid * rpt

        zero16 = jnp.zeros((LANES,), jnp.float32)

        def fill_zero(i, _):
            def inner(j, _):
                rows_v[i, pl.ds(j * LANES, LANES)] = zero16
                return 0
            lax.fori_loop(0, nvec, inner, 0)
            return 0
        lax.fori_loop(0, CH, fill_zero, 0)

        def zchunk(i, _):
            pltpu.sync_copy(rows_v, acc_sh.at[pl.ds(r0 + i * CH, CH)])
            return 0
        lax.fori_loop(0, nch, zchunk, 0)

        def zrows(nf, npart):
            def chunk(ci, rows_n):
                r = r0 + ci * ZCH
                pltpu.sync_copy(x_hbm.at[pl.ds(r, rows_n)],
                                rows_v.at[pl.ds(0, rows_n)])
                pltpu.sync_copy(deg_hbm.at[pl.ds(r, rows_n)],
                                dg0_v.at[pl.ds(0, rows_n)])
                pltpu.sync_copy(deg_hbm.at[pl.ds(n_pad + r, rows_n)],
                                dg1_v.at[pl.ds(0, rows_n)])

                def scale(k, _):
                    dv = _rsqrt16(dg0_v[k, :] + dg1_v[k, :] + 1.0)

                    def vmul(j, _):
                        sl = pl.ds(j * LANES, LANES)
                        rows_v[k, sl] = rows_v[k, sl] * dv
                        return 0
                    lax.fori_loop(0, nvec, vmul, 0)
                    return 0
                lax.fori_loop(0, rows_n, scale, 0)
                pltpu.sync_copy(rows_v.at[pl.ds(0, rows_n)],
                                z_out.at[pl.ds(cid * n_pad + r, rows_n)])

            def floop(ci, _):
                chunk(ci, ZCH)
                return 0
            lax.fori_loop(0, nf, floop, 0)
            if npart:
                chunk(nf, npart)

        @pl.when(sid != NSUB - 1)
        def _():
            zrows(zfull, zpart)

        @pl.when(sid == NSUB - 1)
        def _():
            zrows(lfull, lpart)

        plsc.subcore_barrier()

        zbase = cid * n_pad
        ebase = wid * ept

        def addbase(idx_ref, cnt):
            def vadd(j, _):
                sl = pl.ds(j * LANES, LANES)
                idx_ref[sl] = idx_ref[sl] + zbase
                return 0
            lax.fori_loop(0, cnt // LANES, vadd, 0)

        def batch(i, _):
            off = ebase + i * B
            pltpu.sync_copy(eflat_hbm.at[pl.ds(off, B)], sidx_v)
            pltpu.sync_copy(eflat_hbm.at[pl.ds(n_edges + off, B)], didx_v)
            addbase(sidx_v, B)
            pltpu.async_copy(z_out.at[sidx_v], rows_v, sem).wait()
            pltpu.sync_copy(rows_v, acc_sh.at[didx_v], add=True)
            return 0
        lax.fori_loop(0, nfull, batch, 0)
        if rem:
            off = ebase + nfull * B
            pltpu.sync_copy(eflat_hbm.at[pl.ds(off, rem)], sidxr_v)
            pltpu.sync_copy(eflat_hbm.at[pl.ds(n_edges + off, rem)], didxr_v)
            addbase(sidxr_v, rem)
            pltpu.async_copy(z_out.at[sidxr_v], rowsr_v, sem).wait()
            pltpu.sync_copy(rowsr_v, acc_sh.at[didxr_v], add=True)

        plsc.subcore_barrier()

        def ochunk(i, _):
            r = r0 + i * CH
            pltpu.sync_copy(acc_sh.at[pl.ds(r, CH)], rows_v)
            pltpu.sync_copy(rows_v, s_out.at[pl.ds(cid * n_pad + r, CH)])
            return 0
        lax.fori_loop(0, nch, ochunk, 0)

    return scat_kernel


def _make_sc_combine(n_nodes, n_pad, d):
    rpw = n_pad // NW
    nvec = d // LANES
    last_avail = n_nodes - (NW - 1) * rpw

    def plan(total):
        out, left = [], total
        while left > 0:
            c = min(CH, left)
            out.append(c)
            left -= c
        return out

    chunks = plan(rpw)
    lchunks = plan(last_avail)

    @functools.partial(
        pl.kernel,
        out_type=jax.ShapeDtypeStruct((n_pad, d), jnp.float32),
        mesh=_mesh(),
        scratch_types=[
            pltpu.VMEM((CH, d), jnp.float32),
            pltpu.VMEM((CH, d), jnp.float32),
            pltpu.VMEM((CH, d), jnp.float32),
            pltpu.VMEM((CH, LANES), jnp.float32),
            pltpu.VMEM((CH, LANES), jnp.float32),
        ],
    )
    def comb_kernel(s_hbm, x_hbm, deg_hbm, u_out,
                    s0_v, s1_v, x_v, dg0_v, dg1_v):
        cid = lax.axis_index("c")
        sid = lax.axis_index("s")
        wid = sid * NCORES + cid
        r0 = wid * rpw

        def do_chunk(coff, rows_n):
            r = r0 + coff
            pltpu.sync_copy(s_hbm.at[pl.ds(r, rows_n)],
                            s0_v.at[pl.ds(0, rows_n)])
            pltpu.sync_copy(s_hbm.at[pl.ds(n_pad + r, rows_n)],
                            s1_v.at[pl.ds(0, rows_n)])
            pltpu.sync_copy(x_hbm.at[pl.ds(r, rows_n)],
                            x_v.at[pl.ds(0, rows_n)])
            pltpu.sync_copy(deg_hbm.at[pl.ds(r, rows_n)],
                            dg0_v.at[pl.ds(0, rows_n)])
            pltpu.sync_copy(deg_hbm.at[pl.ds(n_pad + r, rows_n)],
                            dg1_v.at[pl.ds(0, rows_n)])

            def rowop(k, _):
                dv = _rsqrt16(dg0_v[k, :] + dg1_v[k, :] + 1.0)

                def vop(j, _):
                    sl = pl.ds(j * LANES, LANES)
                    s0_v[k, sl] = dv * (s0_v[k, sl] + s1_v[k, sl]
                                        + dv * x_v[k, sl])
                    return 0
                lax.fori_loop(0, nvec, vop, 0)
                return 0
            lax.fori_loop(0, rows_n, rowop, 0)
            pltpu.sync_copy(s0_v.at[pl.ds(0, rows_n)],
                            u_out.at[pl.ds(r, rows_n)])

        @pl.when(wid != NW - 1)
        def _():
            off = 0
            for c in chunks:
                do_chunk(off, c)
                off += c

        @pl.when(wid == NW - 1)
        def _():
            off = 0
            for c in lchunks:
                do_chunk(off, c)
                off += c

    return comb_kernel


def _make_sc_final(n_edges, n_train, d):
    tpt = n_train // NW
    B = 128
    nb = tpt // B
    nj = B // LANES

    @functools.partial(
        pl.kernel,
        out_type=jax.ShapeDtypeStruct((2 * n_train, d), jnp.float32),
        mesh=_mesh(),
        scratch_types=[
            pltpu.VMEM((B,), jnp.int32),
            pltpu.VMEM((B,), jnp.int32),
            pltpu.VMEM((B,), jnp.int32),
            pltpu.VMEM((B, 128), jnp.float32),
            pltpu.SemaphoreType.DMA,
        ],
    )
    def fin_kernel(eflat_hbm, tid_hbm, u_hbm, out_hbm,
                   tid_v, s_v, d_v, r_v, sem):
        cid = lax.axis_index("c")
        sid = lax.axis_index("s")
        wid = sid * NCORES + cid
        base = wid * tpt

        def batch(i, _):
            off = base + i * B
            pltpu.sync_copy(tid_hbm.at[pl.ds(off, B)], tid_v)

            def pick(j, _):
                t16 = tid_v[pl.ds(j * LANES, LANES)]
                pltpu.async_copy(eflat_hbm.at[t16],
                                 s_v.at[pl.ds(j * LANES, LANES)], sem).wait()
                pltpu.async_copy(eflat_hbm.at[t16 + n_edges],
                                 d_v.at[pl.ds(j * LANES, LANES)], sem).wait()
                return 0
            lax.fori_loop(0, nj, pick, 0)

            pltpu.async_copy(u_hbm.at[s_v], r_v, sem).wait()
            pltpu.sync_copy(r_v, out_hbm.at[pl.ds(off, B)])
            pltpu.async_copy(u_hbm.at[d_v], r_v, sem).wait()
            pltpu.sync_copy(r_v, out_hbm.at[pl.ds(n_train + off, B)])
            return 0
        lax.fori_loop(0, nb, batch, 0)

    return fin_kernel


def _tc_head(gg, w1t, b1r, wa, wb, fcbr, n_train, d, blk=2048):
    grid = (n_train // blk,)
    nb = n_train // blk

    def body(g0_ref, g1_ref, w_ref, b_ref, wa_ref, wb_ref, fb_ref, o_ref):
        h0 = jnp.maximum(
            jnp.dot(g0_ref[...], w_ref[...],
                    preferred_element_type=jnp.float32) + b_ref[...], 0.0)
        h1 = jnp.maximum(
            jnp.dot(g1_ref[...], w_ref[...],
                    preferred_element_type=jnp.float32) + b_ref[...], 0.0)
        o_ref[...] = (jnp.dot(h0, wa_ref[...],
                              preferred_element_type=jnp.float32)
                      + jnp.dot(h1, wb_ref[...],
                                preferred_element_type=jnp.float32)
                      + fb_ref[...])

    return pl.pallas_call(
        body,
        grid=grid,
        in_specs=[
            pl.BlockSpec((blk, d), lambda i: (i, 0)),
            pl.BlockSpec((blk, d), lambda i, _nb=nb: (_nb + i, 0)),
            pl.BlockSpec((d, d), lambda i: (0, 0)),
            pl.BlockSpec((1, d), lambda i: (0, 0)),
            pl.BlockSpec((d, LANES), lambda i: (0, 0)),
            pl.BlockSpec((d, LANES), lambda i: (0, 0)),
            pl.BlockSpec((1, LANES), lambda i: (0, 0)),
        ],
        out_specs=pl.BlockSpec((blk, LANES), lambda i: (i, 0)),
        out_shape=jax.ShapeDtypeStruct((n_train, LANES), jnp.float32),
    )(gg, gg, w1t, b1r, wa, wb, fcbr)


def _tc_y(degacc, x2, w1t, n_nodes, d, blk=1024):
    grid = (n_nodes // blk,)

    def body(da_ref, x_ref, w_ref, y_ref):
        dinv = da_ref[:, 0:1]
        xw = jnp.dot(x_ref[...], w_ref[...],
                     preferred_element_type=jnp.float32)
        y_ref[...] = dinv * xw

    return pl.pallas_call(
        body,
        grid=grid,
        in_specs=[
            pl.BlockSpec((blk, LANES), lambda i: (i, 0)),
            pl.BlockSpec((blk, d), lambda i: (i, 0)),
            pl.BlockSpec((d, d), lambda i: (0, 0)),
        ],
        out_specs=pl.BlockSpec((blk, d), lambda i: (i, 0)),
        out_shape=jax.ShapeDtypeStruct((n_nodes, d), jnp.float32),
    )(degacc, x2, w1t)


def _tc_combine(degacc, sacc, y, b1r, wpa, wpb, fcba, n_nodes, d, blk=1024):
    grid = (n_nodes // blk,)

    def body(da_ref, s_ref, y_ref, b_ref, wa_ref, wb_ref, fb_ref,
             a_ref, bt_ref):
        dinv = da_ref[:, 0:1]
        ssum = s_ref[...] + y_ref[...]
        h = jnp.maximum(dinv * ssum + b_ref[...], 0.0)
        a_ref[...] = jnp.dot(h, wa_ref[...],
                             preferred_element_type=jnp.float32) + fb_ref[...]
        bt_ref[...] = jnp.dot(h, wb_ref[...],
                              preferred_element_type=jnp.float32)

    return pl.pallas_call(
        body,
        grid=grid,
        in_specs=[
            pl.BlockSpec((blk, LANES), lambda i: (i, 0)),
            pl.BlockSpec((blk, d), lambda i: (i, 0)),
            pl.BlockSpec((blk, d), lambda i: (i, 0)),
            pl.BlockSpec((1, d), lambda i: (0, 0)),
            pl.BlockSpec((d, LANES), lambda i: (0, 0)),
            pl.BlockSpec((d, LANES), lambda i: (0, 0)),
            pl.BlockSpec((1, LANES), lambda i: (0, 0)),
        ],
        out_specs=[
            pl.BlockSpec((blk, LANES), lambda i: (i, 0)),
            pl.BlockSpec((blk, LANES), lambda i: (i, 0)),
        ],
        out_shape=[
            jax.ShapeDtypeStruct((n_nodes, LANES), jnp.float32),
            jax.ShapeDtypeStruct((n_nodes, LANES), jnp.float32),
        ],
    )(degacc, sacc, y, b1r, wpa, wpb, fcba)


def kernel(x, edge_index, train_edge_id, W1, b1, fcW, fcb):
    n, _, d = x.shape
    e = edge_index.shape[1]
    nt = train_edge_id.shape[0]
    nc = fcW.shape[0]
    n_pad = ((n + NSUB * CH - 1) // (NSUB * CH)) * (NSUB * CH)

    x2 = jnp.pad(x.reshape(n, d), ((0, n_pad - n), (0, 0)))
    e0 = edge_index[0]
    e1 = edge_index[1]

    wpa = jnp.zeros((d, LANES), jnp.float32).at[:, :nc].set(fcW[:, :d].T)
    wpb = jnp.zeros((d, LANES), jnp.float32).at[:, :nc].set(fcW[:, d:].T)
    fcba = jnp.zeros((1, LANES), jnp.float32).at[0, :nc].set(fcb)
    b1r = b1.reshape(1, d)
    w1t = W1.T

    deg = jax.ops.segment_sum(jnp.ones((e,), jnp.float32), e1,
                              num_segments=n_pad)
    dinv = lax.rsqrt(deg + 1.0)
    dinvw = jnp.broadcast_to(dinv[:, None], (n_pad, LANES))
    y = _tc_y(dinvw, x2, w1t, n_pad, d)
    s_sum = jax.ops.segment_sum(y[e0], e1, num_segments=n_pad)
    a_tab, b_tab = _tc_combine(dinvw, s_sum, y, b1r, wpa, wpb, fcba, n_pad, d)
    out16 = a_tab[e0[train_edge_id]] + b_tab[e1[train_edge_id]]
    return out16[:, :nc]
